# baseline (speedup 1.0000x reference)
import functools

import jax
import jax.numpy as jnp
from jax import lax
from jax.experimental import pallas as pl
from jax.experimental.pallas import tpu as pltpu
from jax.experimental.pallas import tpu_sc as plsc

NUM_FIELDS = 26
VOCAB = 100000
EMB = 32
BATCH = 16384

NC, NS, LANES = 2, 16, 16
NW = NC * NS
PACK = 128 // EMB
QUART = VOCAB // PACK
BQ = BATCH // PACK
KB = 512
NKB = BQ // KB
NCHUNKS = NUM_FIELDS * NKB
CROWS = KB * PACK
GATHER = 128
NGATHER = CROWS // GATHER
NVEC = CROWS // LANES

STEP = 5000
OUTKB = 512


def _tc_relayout(tab_t):
    def body(in_ref, out_ref):
        for s in range(QUART // STEP):
            m = jnp.concatenate(
                [
                    in_ref[0, :, j * QUART + s * STEP:j * QUART + (s + 1) * STEP]
                    for j in range(PACK)
                ],
                axis=0,
            )
            out_ref[s * STEP:(s + 1) * STEP, :] = m.T

    return pl.pallas_call(
        body,
        grid=(NUM_FIELDS,),
        in_specs=[pl.BlockSpec((1, EMB, VOCAB), lambda f: (f, 0, 0))],
        out_specs=pl.BlockSpec((QUART, 128), lambda f: (f, 0)),
        out_shape=jax.ShapeDtypeStruct((NUM_FIELDS * QUART, 128), jnp.float32),
        compiler_params=pltpu.CompilerParams(vmem_limit_bytes=100 * 2**20),
    )(tab_t)


def _sc_gather(tab, xt, perm):
    mesh = plsc.VectorSubcoreMesh(core_axis_name="c", subcore_axis_name="s")

    @functools.partial(
        pl.kernel,
        mesh=mesh,
        out_type=jax.ShapeDtypeStruct((NCHUNKS, CROWS, EMB), jnp.float32),
        scratch_types=[
            pltpu.VMEM((CROWS,), jnp.int32),
            pltpu.VMEM((CROWS,), jnp.int32),
            pltpu.VMEM((CROWS,), jnp.int32),
            pltpu.VMEM((CROWS, EMB), jnp.float32),
            pltpu.SemaphoreType.DMA,
        ],
        compiler_params=pltpu.CompilerParams(
            use_tc_tiling_on_sc=False, needs_layout_passes=False
        ),
    )
    def k(xt_hbm, perm_hbm, tab_hbm, out_hbm, xs_v, perm_v, idx_v, rows_v, sem):
        wid = lax.axis_index("s") * NC + lax.axis_index("c")
        pltpu.sync_copy(perm_hbm, perm_v)

        def chunk_body(ci, carry):
            t = wid + ci * NW

            @pl.when(t < NCHUNKS)
            def _():
                f = t >> 3
                kb = t & (NKB - 1)
                for j in range(PACK):
                    pltpu.sync_copy(
                        xt_hbm.at[f, pl.ds(j * BQ + kb * KB, KB)],
                        xs_v.at[pl.ds(j * KB, KB)],
                    )

                def idx_body(i, c2):
                    o = pl.multiple_of(i * LANES, LANES)
                    pv = perm_v[pl.ds(o, LANES)]
                    v = plsc.load_gather(xs_v, [pv])
                    quart = (
                        jnp.where(v >= QUART, 1, 0)
                        + jnp.where(v >= 2 * QUART, 1, 0)
                        + jnp.where(v >= 3 * QUART, 1, 0)
                    )
                    idx_v[pl.ds(o, LANES)] = (
                        f * VOCAB + (v - quart * QUART) * PACK + quart
                    )
                    return c2

                lax.fori_loop(0, NVEC, idx_body, 0)

                cps = [
                    pltpu.async_copy(
                        tab_hbm.at[idx_v.at[pl.ds(g * GATHER, GATHER)]],
                        rows_v.at[pl.ds(g * GATHER, GATHER)],
                        sem,
                    )
                    for g in range(NGATHER)
                ]
                for cp in cps:
                    cp.wait()
                pltpu.sync_copy(rows_v, out_hbm.at[t])

            return carry

        lax.fori_loop(0, (NCHUNKS + NW - 1) // NW, chunk_body, 0)

    return k(xt, perm, tab)


def _tc_untile(packed):
    def body(in_ref, out_ref):
        j = pl.program_id(2)
        t = in_ref[...].T
        for jj in range(PACK):
            @pl.when(j == jj)
            def _(jj=jj):
                out_ref[...] = t[jj * EMB:(jj + 1) * EMB]

    return pl.pallas_call(
        body,
        grid=(NUM_FIELDS, BQ // OUTKB, PACK),
        in_specs=[
            pl.BlockSpec((OUTKB, 128), lambda f, kb, j: (f * (BQ // OUTKB) + kb, 0)),
        ],
        out_specs=pl.BlockSpec(
            (EMB, OUTKB),
            lambda f, kb, j: (f, j * (BQ // OUTKB) + kb),
        ),
        out_shape=jax.ShapeDtypeStruct((NUM_FIELDS * EMB, BATCH), jnp.float32),
    )(packed)


def kernel(x, tables):
    tab_flat = _tc_relayout(tables.transpose(0, 2, 1)).reshape(
        NUM_FIELDS * VOCAB, EMB
    )
    xt = x.astype(jnp.int32).T
    perm = (
        (jnp.arange(CROWS, dtype=jnp.int32) % PACK) * KB
        + jnp.arange(CROWS, dtype=jnp.int32) // PACK
    )
    out_packed = _sc_gather(tab_flat, xt, perm).reshape(
        NUM_FIELDS * BQ, PACK * EMB
    )
    out_t = _tc_untile(out_packed)
    return out_t.T


# speedup vs baseline: 1.0096x; 1.0096x over previous
import functools

import jax
import jax.numpy as jnp
from jax import lax
from jax.experimental import pallas as pl
from jax.experimental.pallas import tpu as pltpu
from jax.experimental.pallas import tpu_sc as plsc

NUM_FIELDS = 26
VOCAB = 100000
EMB = 32
BATCH = 16384

NC, NS, LANES = 2, 16, 16
NW = NC * NS
PACK = 128 // EMB
QUART = VOCAB // PACK
BQ = BATCH // PACK
KB = 256
NKB = BQ // KB
NCHUNKS = NUM_FIELDS * NKB
CROWS = KB * PACK
GATHER = 128
NGATHER = CROWS // GATHER
NVEC = CROWS // LANES
NKSTEP = KB // LANES

STEP = 5000


def _tc_relayout(tab_t):
    def body(in_ref, out_ref):
        for s in range(QUART // STEP):
            m = jnp.concatenate(
                [
                    in_ref[0, :, j * QUART + s * STEP:j * QUART + (s + 1) * STEP]
                    for j in range(PACK)
                ],
                axis=0,
            )
            out_ref[s * STEP:(s + 1) * STEP, :] = m.T

    return pl.pallas_call(
        body,
        grid=(NUM_FIELDS,),
        in_specs=[pl.BlockSpec((1, EMB, VOCAB), lambda f: (f, 0, 0))],
        out_specs=pl.BlockSpec((QUART, 128), lambda f: (f, 0)),
        out_shape=jax.ShapeDtypeStruct((NUM_FIELDS * QUART, 128), jnp.float32),
        compiler_params=pltpu.CompilerParams(vmem_limit_bytes=100 * 2**20),
    )(tab_t)


def _sc_gather(tab, xt, perm):
    mesh = plsc.VectorSubcoreMesh(core_axis_name="c", subcore_axis_name="s")

    @functools.partial(
        pl.kernel,
        mesh=mesh,
        out_type=jax.ShapeDtypeStruct((NUM_FIELDS * EMB, BATCH), jnp.float32),
        scratch_types=[
            pltpu.VMEM((CROWS,), jnp.int32),
            pltpu.VMEM((CROWS,), jnp.int32),
            pltpu.VMEM((CROWS,), jnp.int32),
            pltpu.VMEM((CROWS, EMB), jnp.float32),
            pltpu.VMEM((PACK * EMB, KB), jnp.float32),
            pltpu.SemaphoreType.DMA,
        ],
        compiler_params=pltpu.CompilerParams(
            use_tc_tiling_on_sc=False, needs_layout_passes=False
        ),
    )
    def k(xt_hbm, perm_hbm, tab_hbm, out_hbm,
          xs_v, perm_v, idx_v, rows_v, arr_v, sem):
        wid = lax.axis_index("s") * NC + lax.axis_index("c")
        pltpu.sync_copy(perm_hbm, perm_v)
        lane = lax.iota(jnp.int32, LANES)

        def chunk_body(ci, carry):
            t = wid + ci * NW

            @pl.when(t < NCHUNKS)
            def _():
                f = t >> 4
                kb = t & (NKB - 1)
                for j in range(PACK):
                    pltpu.sync_copy(
                        xt_hbm.at[f, pl.ds(j * BQ + kb * KB, KB)],
                        xs_v.at[pl.ds(j * KB, KB)],
                    )

                def idx_body(i, c2):
                    o = pl.multiple_of(i * LANES, LANES)
                    pv = perm_v[pl.ds(o, LANES)]
                    v = plsc.load_gather(xs_v, [pv])
                    quart = (
                        jnp.where(v >= QUART, 1, 0)
                        + jnp.where(v >= 2 * QUART, 1, 0)
                        + jnp.where(v >= 3 * QUART, 1, 0)
                    )
                    idx_v[pl.ds(o, LANES)] = (
                        f * VOCAB + (v - quart * QUART) * PACK + quart
                    )
                    return c2

                lax.fori_loop(0, NVEC, idx_body, 0)

                cps = [
                    pltpu.async_copy(
                        tab_hbm.at[idx_v.at[pl.ds(g * GATHER, GATHER)]],
                        rows_v.at[pl.ds(g * GATHER, GATHER)],
                        sem,
                    )
                    for g in range(NGATHER)
                ]
                for cp in cps:
                    cp.wait()

                def row_body(c2, carry2):
                    j = c2 >> 5
                    e = c2 & (EMB - 1)

                    def kstep(s, carry3):
                        ri = lane * PACK + s * (LANES * PACK) + j
                        ei = jnp.zeros((LANES,), jnp.int32) + e
                        vals = plsc.load_gather(rows_v, [ri, ei])
                        o3 = pl.multiple_of(s * LANES, LANES)
                        arr_v[c2, pl.ds(o3, LANES)] = vals
                        return carry3

                    lax.fori_loop(0, NKSTEP, kstep, 0)
                    return carry2

                lax.fori_loop(0, PACK * EMB, row_body, 0)

                for j in range(PACK):
                    pltpu.sync_copy(
                        arr_v.at[pl.ds(j * EMB, EMB)],
                        out_hbm.at[
                            pl.ds(f * EMB, EMB),
                            pl.ds(j * BQ + kb * KB, KB),
                        ],
                    )

            return carry

        lax.fori_loop(0, (NCHUNKS + NW - 1) // NW, chunk_body, 0)

    return k(xt, perm, tab)


def kernel(x, tables):
    tab_flat = _tc_relayout(tables.transpose(0, 2, 1)).reshape(
        NUM_FIELDS * VOCAB, EMB
    )
    xt = x.astype(jnp.int32).T
    perm = (
        (jnp.arange(CROWS, dtype=jnp.int32) % PACK) * KB
        + jnp.arange(CROWS, dtype=jnp.int32) // PACK
    )
    out_t = _sc_gather(tab_flat, xt, perm)
    return out_t.T


# speedup vs baseline: 1.7277x; 1.7112x over previous
import functools

import jax
import jax.numpy as jnp
from jax import lax
from jax.experimental import pallas as pl
from jax.experimental.pallas import tpu as pltpu
from jax.experimental.pallas import tpu_sc as plsc

NUM_FIELDS = 26
VOCAB = 100000
EMB = 32
BATCH = 16384

N_ROWS = BATCH * NUM_FIELDS
NC, NS, LANES = 2, 16, 16
NW = NC * NS
ROWS_PER_W = N_ROWS // NW
CHUNK = 1664
NCHUNKS = ROWS_PER_W // CHUNK
GATHER = 128
NGATHER = CHUNK // GATHER
NVEC = CHUNK // LANES
NROWBLK = N_ROWS // CHUNK

PACK = 128 // EMB


def _tc_relayout(tab_t):
    QUART = VOCAB // PACK
    STEP = 5000

    def body(in_ref, out_ref):
        for s in range(QUART // STEP):
            m = jnp.concatenate(
                [
                    in_ref[0, :, j * QUART + s * STEP:j * QUART + (s + 1) * STEP]
                    for j in range(PACK)
                ],
                axis=0,
            )
            out_ref[s * STEP:(s + 1) * STEP, :] = m.T

    return pl.pallas_call(
        body,
        grid=(NUM_FIELDS,),
        in_specs=[
            pl.BlockSpec((1, EMB, VOCAB), lambda f: (f, 0, 0)),
        ],
        out_specs=pl.BlockSpec((VOCAB // PACK, 128), lambda f: (f, 0)),
        out_shape=jax.ShapeDtypeStruct(
            (NUM_FIELDS * VOCAB // PACK, 128), jnp.float32
        ),
        compiler_params=pltpu.CompilerParams(vmem_limit_bytes=100 * 2**20),
    )(tab_t)


def _sc_gather(tab, x2d, offs):
    mesh = plsc.VectorSubcoreMesh(core_axis_name="c", subcore_axis_name="s")

    @functools.partial(
        pl.kernel,
        mesh=mesh,
        out_type=jax.ShapeDtypeStruct((NROWBLK, CHUNK, EMB), jnp.float32),
        scratch_types=[
            pltpu.VMEM((CHUNK,), jnp.int32),
            pltpu.VMEM((CHUNK,), jnp.int32),
            pltpu.VMEM((CHUNK,), jnp.int32),
            pltpu.VMEM((CHUNK, EMB), jnp.float32),
            pltpu.SemaphoreType.DMA,
        ],
        compiler_params=pltpu.CompilerParams(use_tc_tiling_on_sc=False),
    )
    def k(x_hbm, offs_hbm, tab_hbm, out_hbm, x_v, offs_v, idx_v, rows_v, sem):
        wid = lax.axis_index("s") * NC + lax.axis_index("c")
        pltpu.sync_copy(offs_hbm, offs_v)

        def chunk_body(ci, carry):
            blk = wid * NCHUNKS + ci
            pltpu.sync_copy(x_hbm.at[blk], x_v)

            def add_body(i, c2):
                o = pl.multiple_of(i * LANES, LANES)
                v = x_v[pl.ds(o, LANES)]
                quart = (
                    jnp.where(v >= (VOCAB // PACK), 1, 0)
                    + jnp.where(v >= 2 * (VOCAB // PACK), 1, 0)
                    + jnp.where(v >= 3 * (VOCAB // PACK), 1, 0)
                )
                idx_v[pl.ds(o, LANES)] = (
                    offs_v[pl.ds(o, LANES)]
                    + (v - quart * (VOCAB // PACK)) * PACK
                    + quart
                )
                return c2

            lax.fori_loop(0, NVEC, add_body, 0)

            cps = [
                pltpu.async_copy(
                    tab_hbm.at[idx_v.at[pl.ds(g * GATHER, GATHER)]],
                    rows_v.at[pl.ds(g * GATHER, GATHER)],
                    sem,
                )
                for g in range(NGATHER)
            ]
            for cp in cps:
                cp.wait()
            pltpu.sync_copy(rows_v, out_hbm.at[blk])
            return carry

        lax.fori_loop(0, NCHUNKS, chunk_body, 0)

    return k(x2d, offs, tab)


def kernel(x, tables):
    tab_flat = _tc_relayout(tables.transpose(0, 2, 1)).reshape(
        NUM_FIELDS * VOCAB, EMB
    )
    xf = x.astype(jnp.int32).reshape(NROWBLK, CHUNK)
    offs = (jnp.arange(CHUNK, dtype=jnp.int32) % NUM_FIELDS) * VOCAB
    out = _sc_gather(tab_flat, xf, offs)
    return out.reshape(BATCH, NUM_FIELDS * EMB)
